# Initial kernel scaffold; baseline (speedup 1.0000x reference)
import jax
import jax.numpy as jnp
from jax.experimental import pallas as pl


def kernel(z, codebook):
    raise NotImplementedError("write your pallas kernel here")


# speedup vs baseline: 3.0237x; 3.0237x over previous
import jax
import jax.numpy as jnp
from jax.experimental import pallas as pl


_DIM = 2048
_H = 8
_M = 1024
_DH = _DIM // _H
_TB = 256


def _body(z_ref, z2_ref, c2_ref, cbt_ref, cb_ref, zq_ref, idx_ref, md_ref):
    iot = jax.lax.broadcasted_iota(jnp.int32, (_TB, _M), 1)
    for h in range(_H):
        zb = z_ref[:, h * _DH:(h + 1) * _DH]
        cross = jax.lax.dot_general(
            zb, cbt_ref[h],
            dimension_numbers=(((1,), (0,)), ((), ())),
            preferred_element_type=jnp.float32)
        z2 = z2_ref[:, h:h + 1]
        c2 = c2_ref[h:h + 1, :]
        d = (z2 + c2) - 2.0 * cross
        d = jnp.maximum(d, 0.0)
        m = jnp.min(d, axis=1, keepdims=True)
        idx = jnp.min(jnp.where(d == m, iot, _M), axis=1)
        onehot = (iot == idx[:, None]).astype(jnp.float32)
        e = jax.lax.dot_general(
            onehot, cb_ref[h],
            dimension_numbers=(((1,), (0,)), ((), ())),
            preferred_element_type=jnp.float32,
            precision=jax.lax.Precision.HIGHEST)
        zq_ref[:, h * _DH:(h + 1) * _DH] = e
        idx_ref[:, h:h + 1] = idx[:, None]
        md_ref[:, h:h + 1] = m


def kernel(z, codebook):
    Bb, Ll, dim = z.shape
    N = Bb * Ll
    zr = z.reshape(N, dim)
    z_h = z.reshape(N, _H, _DH)
    z2 = jnp.sum(z_h ** 2, axis=-1)
    c2 = jnp.sum(codebook ** 2, axis=-1)
    cbt = codebook.transpose(0, 2, 1)

    grid = (N // _TB,)
    zq, idx, md = pl.pallas_call(
        _body,
        grid=grid,
        in_specs=[
            pl.BlockSpec((_TB, dim), lambda i: (i, 0)),
            pl.BlockSpec((_TB, _H), lambda i: (i, 0)),
            pl.BlockSpec((_H, _M), lambda i: (0, 0)),
            pl.BlockSpec((_H, _DH, _M), lambda i: (0, 0, 0)),
            pl.BlockSpec((_H, _M, _DH), lambda i: (0, 0, 0)),
        ],
        out_specs=[
            pl.BlockSpec((_TB, dim), lambda i: (i, 0)),
            pl.BlockSpec((_TB, _H), lambda i: (i, 0)),
            pl.BlockSpec((_TB, _H), lambda i: (i, 0)),
        ],
        out_shape=[
            jax.ShapeDtypeStruct((N, dim), jnp.float32),
            jax.ShapeDtypeStruct((N, _H), jnp.int32),
            jax.ShapeDtypeStruct((N, _H), jnp.float32),
        ],
    )(zr, z2, c2, cbt, codebook)

    return (zq.reshape(Bb, Ll, dim),
            idx.reshape(Bb, Ll, _H),
            md.reshape(Bb, Ll, _H))


# speedup vs baseline: 3.8106x; 1.2602x over previous
import functools

import jax
import jax.numpy as jnp
from jax import lax
from jax.experimental import pallas as pl
from jax.experimental.pallas import tpu as pltpu
from jax.experimental.pallas import tpu_sc as plsc


_DIM = 2048
_H = 8
_M = 1024
_DH = _DIM // _H
_TB = 256


def _tc_body(z_ref, z2_ref, c2_ref, cbt_ref, idx_ref, gidx_ref, md_ref):
    iot = jax.lax.broadcasted_iota(jnp.int32, (_TB, _M), 1)
    for h in range(_H):
        zb = z_ref[:, h * _DH:(h + 1) * _DH]
        cross = jax.lax.dot_general(
            zb, cbt_ref[h],
            dimension_numbers=(((1,), (0,)), ((), ())),
            preferred_element_type=jnp.float32)
        z2 = z2_ref[:, h:h + 1]
        c2 = c2_ref[h:h + 1, :]
        d = (z2 + c2) - 2.0 * cross
        d = jnp.maximum(d, 0.0)
        m = jnp.min(d, axis=1, keepdims=True)
        idx = jnp.min(jnp.where(d == m, iot, _M), axis=1)
        idx_ref[:, h:h + 1] = idx[:, None]
        gidx_ref[:, h:h + 1] = idx[:, None] + (h * _M)
        md_ref[:, h:h + 1] = m


def _tc_call(zr, z2, c2, cbt):
    N = zr.shape[0]
    grid = (N // _TB,)
    return pl.pallas_call(
        _tc_body,
        grid=grid,
        in_specs=[
            pl.BlockSpec((_TB, _DIM), lambda i: (i, 0)),
            pl.BlockSpec((_TB, _H), lambda i: (i, 0)),
            pl.BlockSpec((_H, _M), lambda i: (0, 0)),
            pl.BlockSpec((_H, _DH, _M), lambda i: (0, 0, 0)),
        ],
        out_specs=[
            pl.BlockSpec((_TB, _H), lambda i: (i, 0)),
            pl.BlockSpec((_TB, _H), lambda i: (i, 0)),
            pl.BlockSpec((_TB, _H), lambda i: (i, 0)),
        ],
        out_shape=[
            jax.ShapeDtypeStruct((N, _H), jnp.int32),
            jax.ShapeDtypeStruct((N, _H), jnp.int32),
            jax.ShapeDtypeStruct((N, _H), jnp.float32),
        ],
    )(zr, z2, c2, cbt)


_SC_CHUNK = 128


def _sc_gather(table, gidx):
    R = gidx.shape[0]
    info = plsc.get_sparse_core_info()
    nw = info.num_cores * info.num_subcores
    rows_per_w = R // nw
    n_chunks = rows_per_w // _SC_CHUNK
    mesh = plsc.VectorSubcoreMesh(core_axis_name="c", subcore_axis_name="s")

    @functools.partial(
        pl.kernel, mesh=mesh,
        out_type=jax.ShapeDtypeStruct((R, _DH), jnp.float32),
        scratch_types=[
            pltpu.VMEM((_SC_CHUNK,), jnp.int32),
            pltpu.VMEM((_SC_CHUNK, _DH), jnp.float32),
            pltpu.SemaphoreType.DMA,
        ],
    )
    def k(table_hbm, gidx_hbm, out_hbm, idx_v, rows_v, sem):
        wid = lax.axis_index("s") * info.num_cores + lax.axis_index("c")
        base = wid * rows_per_w
        for c in range(n_chunks):
            off = base + c * _SC_CHUNK
            pltpu.sync_copy(gidx_hbm.at[pl.ds(off, _SC_CHUNK)], idx_v)
            pltpu.async_copy(table_hbm.at[idx_v], rows_v, sem).wait()
            pltpu.sync_copy(rows_v, out_hbm.at[pl.ds(off, _SC_CHUNK)])

    return k(table, gidx)


def kernel(z, codebook):
    Bb, Ll, dim = z.shape
    N = Bb * Ll
    zr = z.reshape(N, dim)
    z_h = z.reshape(N, _H, _DH)
    z2 = jnp.sum(z_h ** 2, axis=-1)
    c2 = jnp.sum(codebook ** 2, axis=-1)
    cbt = codebook.transpose(0, 2, 1)

    idx, gidx, md = _tc_call(zr, z2, c2, cbt)

    table = codebook.reshape(_H * _M, _DH)
    zq = _sc_gather(table, gidx.reshape(N * _H))

    return (zq.reshape(Bb, Ll, dim),
            idx.reshape(Bb, Ll, _H),
            md.reshape(Bb, Ll, _H))
